# baseline (speedup 1.0000x reference)
import functools

import jax
import jax.numpy as jnp
from jax import lax
from jax.experimental import pallas as pl
from jax.experimental.pallas import tpu as pltpu
from jax.experimental.pallas import tpu_sc as plsc

NUM_STOCKS = 1000000
EMBED_DIM = 64
BATCH = 16384
SEQ_LEN = 50

NC = 2
NS = 16
NW = NC * NS

BW = 128
UNITS_B = BATCH // BW
B_PER_W = BATCH // NW
CPW = B_PER_W // BW
NUNIT = SEQ_LEN * CPW
NBUF = 2
L = 16
NG = BW // L


def _gather_kernel(ids_t, tab2):
    mesh = plsc.VectorSubcoreMesh(core_axis_name="c", subcore_axis_name="s")

    @functools.partial(
        pl.kernel,
        mesh=mesh,
        out_type=jax.ShapeDtypeStruct((SEQ_LEN, EMBED_DIM, BATCH), jnp.float32),
        scratch_types=[
            pltpu.VMEM((SEQ_LEN, B_PER_W), jnp.int32),
            pltpu.VMEM((NBUF, BW), jnp.int32),
            pltpu.VMEM((NBUF, BW), jnp.int32),
            pltpu.VMEM((NBUF, BW, BW), jnp.float32),
            pltpu.VMEM((NBUF, EMBED_DIM, BW), jnp.float32),
            [pltpu.SemaphoreType.DMA] * NBUF,
            [pltpu.SemaphoreType.DMA] * NBUF,
        ],
        compiler_params=pltpu.CompilerParams(needs_layout_passes=False),
    )
    def k(ids_hbm, tab_hbm, out_hbm, ids_v, idxp_v, half_v, pair_v, out_v,
          gsem, wsem):
        wid = lax.axis_index("s") * NC + lax.axis_index("c")
        b_base = wid * B_PER_W

        pltpu.sync_copy(ids_hbm.at[:, pl.ds(b_base, B_PER_W)], ids_v)

        def prep_and_fire(u, b):
            s = u // CPW
            c = lax.rem(u, CPW)
            for g in range(NG):
                ids_g = ids_v[s, pl.ds(c * BW + g * L, L)]
                idxp_v[b, pl.ds(g * L, L)] = lax.shift_right_logical(ids_g, 1)
                half_v[b, pl.ds(g * L, L)] = lax.shift_left(
                    lax.bitwise_and(ids_g, 1), 6)
            pltpu.async_copy(tab_hbm.at[idxp_v.at[b]], pair_v.at[b], gsem[b])

        def gather_wait(b):
            pltpu.make_async_copy(
                tab_hbm.at[idxp_v.at[b]], pair_v.at[b], gsem[b]).wait()

        def write_wait(u, b):
            s = u // CPW
            c = lax.rem(u, CPW)
            pltpu.make_async_copy(
                out_v.at[b],
                out_hbm.at[s, :, pl.ds(b_base + c * BW, BW)],
                wsem[b],
            ).wait()

        for b in range(NBUF):
            prep_and_fire(b, b)

        @pl.loop(0, NUNIT, step=NBUF)
        def unit_loop(u0):
            for b in range(NBUF):
                u = u0 + b
                gather_wait(b)
                for g in range(NG):
                    rows = lax.iota(jnp.int32, L) + g * L
                    halfs = half_v[b, pl.ds(g * L, L)]
                    for d in range(EMBED_DIM):
                        v = plsc.load_gather(pair_v.at[b], [rows, halfs + d])
                        out_v[b, d, pl.ds(g * L, L)] = v
                @pl.when(u >= NBUF)
                def _():
                    write_wait(u - NBUF, b)
                s = u // CPW
                c = lax.rem(u, CPW)
                pltpu.async_copy(
                    out_v.at[b],
                    out_hbm.at[s, :, pl.ds(b_base + c * BW, BW)],
                    wsem[b],
                )
                @pl.when(u + NBUF < NUNIT)
                def _():
                    prep_and_fire(u + NBUF, b)

        for b in range(NBUF):
            write_wait(NUNIT - NBUF + b, b)

    return k(ids_t, tab2)


def kernel(stock_ids, table):
    ids_t = stock_ids.T.astype(jnp.int32)
    tab2 = table.reshape(NUM_STOCKS // 2, 2 * EMBED_DIM)
    out_p = _gather_kernel(ids_t, tab2)
    return out_p.transpose(2, 0, 1)
